# Initial kernel scaffold; baseline (speedup 1.0000x reference)
import jax
import jax.numpy as jnp
from jax.experimental import pallas as pl


def kernel(x, pe):
    raise NotImplementedError("write your pallas kernel here")


# speedup vs baseline: 2.8228x; 2.8228x over previous
import jax
import jax.numpy as jnp
from jax.experimental import pallas as pl


def _add_body(x_ref, pe_ref, o_ref):
    o_ref[...] = x_ref[...] + pe_ref[...]


def kernel(x, pe):
    batch, seq_len, d_model = x.shape
    blk = 512
    out = pl.pallas_call(
        _add_body,
        grid=(seq_len // blk, batch),
        in_specs=[
            pl.BlockSpec((1, blk, d_model), lambda s, b: (b, s, 0)),
            pl.BlockSpec((blk, d_model), lambda s, b: (s, 0)),
        ],
        out_specs=pl.BlockSpec((1, blk, d_model), lambda s, b: (b, s, 0)),
        out_shape=jax.ShapeDtypeStruct(x.shape, x.dtype),
    )(x, pe[:seq_len])
    return out
